# baseline (speedup 1.0000x reference)
import jax
import jax.numpy as jnp
from jax import lax
from jax.experimental import pallas as pl
from jax.experimental.pallas import tpu as pltpu
from jax.experimental.pallas import tpu_sc as plsc

B = 16384
D = 128
NC = 2
NS = 16
NW = NC * NS
CH = 128
H = 4
BH = B // H
BPW = BH // NW
NCHUNK = BPW // CH


def _lut_body(at_ref, wa_ref, ba_ref, fw_ref, bf_ref, out_ref):
    feat = jnp.dot(at_ref[...], wa_ref[...], preferred_element_type=jnp.float32)
    feat = feat + ba_ref[...]
    w2 = fw_ref[D:, :]
    out_ref[...] = jnp.dot(feat, w2, preferred_element_type=jnp.float32) + bf_ref[...]


def _fuse_body(semb_ref, fw_ref, acont_ref, out_ref):
    w1 = fw_ref[:D, :]
    out_ref[...] = (
        jnp.dot(semb_ref[...], w1, preferred_element_type=jnp.float32)
        + acont_ref[...]
    )


def _fuse_body_aliased(prev_ref, semb_ref, fw_ref, acont_ref, out_ref):
    del prev_ref
    w1 = fw_ref[:D, :]
    out_ref[...] = (
        jnp.dot(semb_ref[...], w1, preferred_element_type=jnp.float32)
        + acont_ref[...]
    )


def _make_sc_gather(h):
    def _sc_gather(node_hbm, attr_hbm, stab_hbm, alut_hbm, semb_hbm, acont_hbm,
                   nidx_v, aidx_v, srow0, srow1, arow0, arow1,
                   sem_s0, sem_s1, sem_a0, sem_a1):
        wid = lax.axis_index("s") * NC + lax.axis_index("c")
        base = wid * BPW
        srow = (srow0, srow1)
        arow = (arow0, arow1)
        sem_s = (sem_s0, sem_s1)
        sem_a = (sem_a0, sem_a1)
        pltpu.sync_copy(node_hbm.at[h * NW + wid], nidx_v)
        pltpu.sync_copy(attr_hbm.at[h * NW + wid], aidx_v)
        copies = [None, None]
        for j in range(NCHUNK + 1):
            if j < NCHUNK:
                s = j % 2
                cs = pltpu.async_copy(stab_hbm.at[nidx_v.at[j]], srow[s], sem_s[s])
                ca = pltpu.async_copy(alut_hbm.at[aidx_v.at[j]], arow[s], sem_a[s])
                copies[s] = (cs, ca)
            if j >= 1:
                p = (j - 1) % 2
                cs, ca = copies[p]
                cs.wait()
                ca.wait()
                off = base + (j - 1) * CH
                pltpu.sync_copy(srow[p], semb_hbm.at[pl.ds(off, CH)])
                pltpu.sync_copy(arow[p], acont_hbm.at[pl.ds(off, CH)])

    return _sc_gather


def kernel(node_ids, attrs, struct_table, attr_table, attr_fc_w, attr_fc_b, fusion_w, fusion_b):
    attr_lut = pl.pallas_call(
        _lut_body,
        out_shape=jax.ShapeDtypeStruct((1001, D), jnp.float32),
    )(attr_table, attr_fc_w, attr_fc_b.reshape(1, D), fusion_w,
      fusion_b.reshape(1, D))

    mesh = plsc.VectorSubcoreMesh(core_axis_name="c", subcore_axis_name="s",
                                  num_cores=NC, num_subcores=NS)
    node3d = node_ids.reshape(H * NW, NCHUNK, CH)
    attr3d = attrs.reshape(H * NW, NCHUNK, CH)
    scratch = [
        pltpu.VMEM((NCHUNK, CH), jnp.int32),
        pltpu.VMEM((NCHUNK, CH), jnp.int32),
        pltpu.VMEM((CH, D), jnp.float32),
        pltpu.VMEM((CH, D), jnp.float32),
        pltpu.VMEM((CH, D), jnp.float32),
        pltpu.VMEM((CH, D), jnp.float32),
        pltpu.SemaphoreType.DMA,
        pltpu.SemaphoreType.DMA,
        pltpu.SemaphoreType.DMA,
        pltpu.SemaphoreType.DMA,
    ]
    halves = [
        pl.kernel(
            _make_sc_gather(h),
            out_type=[
                jax.ShapeDtypeStruct((BH, D), jnp.float32),
                jax.ShapeDtypeStruct((BH, D), jnp.float32),
            ],
            mesh=mesh,
            scratch_types=scratch,
        )(node3d, attr3d, struct_table, attr_lut)
        for h in range(H)
    ]

    BLK = 2048
    nblk = BH // BLK
    semb0, acont0 = halves[0]
    out = pl.pallas_call(
        _fuse_body,
        grid=(nblk,),
        in_specs=[
            pl.BlockSpec((BLK, D), lambda i: (i, 0)),
            pl.BlockSpec((2 * D, D), lambda i: (0, 0)),
            pl.BlockSpec((BLK, D), lambda i: (i, 0)),
        ],
        out_specs=pl.BlockSpec((BLK, D), lambda i: (i, 0)),
        out_shape=jax.ShapeDtypeStruct((B, D), jnp.float32),
    )(semb0, fusion_w, acont0)
    for h in range(1, H):
        semb_h, acont_h = halves[h]
        out = pl.pallas_call(
            _fuse_body_aliased,
            grid=(nblk,),
            in_specs=[
                pl.BlockSpec(memory_space=pltpu.MemorySpace.HBM),
                pl.BlockSpec((BLK, D), lambda i: (i, 0)),
                pl.BlockSpec((2 * D, D), lambda i: (0, 0)),
                pl.BlockSpec((BLK, D), lambda i: (i, 0)),
            ],
            out_specs=pl.BlockSpec((BLK, D), lambda i, h=h: (i + h * nblk, 0)),
            out_shape=jax.ShapeDtypeStruct((B, D), jnp.float32),
            input_output_aliases={0: 0},
        )(out, semb_h, fusion_w, acont_h)
    return out


# speedup vs baseline: 1.2371x; 1.2371x over previous
import jax
import jax.numpy as jnp
from jax import lax
from jax.experimental import pallas as pl
from jax.experimental.pallas import tpu as pltpu
from jax.experimental.pallas import tpu_sc as plsc

B = 16384
D = 128
NC = 2
NS = 16
NW = NC * NS
CH = 128
H = 2
BH = B // H
BPW = BH // NW
NCHUNK = BPW // CH


def _lut_body(at_ref, wa_ref, ba_ref, fw_ref, bf_ref, out_ref):
    feat = jnp.dot(at_ref[...], wa_ref[...], preferred_element_type=jnp.float32)
    feat = feat + ba_ref[...]
    w2 = fw_ref[D:, :]
    out_ref[...] = jnp.dot(feat, w2, preferred_element_type=jnp.float32) + bf_ref[...]


def _fuse_body(semb_ref, fw_ref, acont_ref, out_ref):
    w1 = fw_ref[:D, :]
    out_ref[...] = (
        jnp.dot(semb_ref[...], w1, preferred_element_type=jnp.float32)
        + acont_ref[...]
    )


def _fuse_body_aliased(prev_ref, semb_ref, fw_ref, acont_ref, out_ref):
    del prev_ref
    w1 = fw_ref[:D, :]
    out_ref[...] = (
        jnp.dot(semb_ref[...], w1, preferred_element_type=jnp.float32)
        + acont_ref[...]
    )


def _make_sc_gather(h):
    def _sc_gather(node_hbm, attr_hbm, stab_hbm, alut_hbm, semb_hbm, acont_hbm,
                   nidx_v, aidx_v, srow0, srow1, arow0, arow1, alut_sh,
                   sem_s0, sem_s1, sem_a0, sem_a1):
        wid = lax.axis_index("s") * NC + lax.axis_index("c")
        sid = lax.axis_index("s")
        base = wid * BPW
        srow = (srow0, srow1)
        arow = (arow0, arow1)
        sem_s = (sem_s0, sem_s1)
        sem_a = (sem_a0, sem_a1)
        @pl.when(sid == 0)
        def _stage():
            pltpu.sync_copy(alut_hbm, alut_sh)

        pltpu.sync_copy(node_hbm.at[h * NW + wid], nidx_v)
        pltpu.sync_copy(attr_hbm.at[h * NW + wid], aidx_v)
        plsc.subcore_barrier()
        copies = [None, None]
        for j in range(NCHUNK + 1):
            if j < NCHUNK:
                s = j % 2
                cs = pltpu.async_copy(stab_hbm.at[nidx_v.at[j]], srow[s], sem_s[s])
                ca = pltpu.async_copy(alut_sh.at[aidx_v.at[j]], arow[s], sem_a[s])
                copies[s] = (cs, ca)
            if j >= 1:
                p = (j - 1) % 2
                cs, ca = copies[p]
                cs.wait()
                ca.wait()
                off = base + (j - 1) * CH
                pltpu.sync_copy(srow[p], semb_hbm.at[pl.ds(off, CH)])
                pltpu.sync_copy(arow[p], acont_hbm.at[pl.ds(off, CH)])

    return _sc_gather


def kernel(node_ids, attrs, struct_table, attr_table, attr_fc_w, attr_fc_b, fusion_w, fusion_b):
    attr_lut = pl.pallas_call(
        _lut_body,
        out_shape=jax.ShapeDtypeStruct((1001, D), jnp.float32),
    )(attr_table, attr_fc_w, attr_fc_b.reshape(1, D), fusion_w,
      fusion_b.reshape(1, D))

    mesh = plsc.VectorSubcoreMesh(core_axis_name="c", subcore_axis_name="s",
                                  num_cores=NC, num_subcores=NS)
    node3d = node_ids.reshape(H * NW, NCHUNK, CH)
    attr3d = attrs.reshape(H * NW, NCHUNK, CH)
    scratch = [
        pltpu.VMEM((NCHUNK, CH), jnp.int32),
        pltpu.VMEM((NCHUNK, CH), jnp.int32),
        pltpu.VMEM((CH, D), jnp.float32),
        pltpu.VMEM((CH, D), jnp.float32),
        pltpu.VMEM((CH, D), jnp.float32),
        pltpu.VMEM((CH, D), jnp.float32),
        pltpu.VMEM_SHARED((1001, D), jnp.float32),
        pltpu.SemaphoreType.DMA,
        pltpu.SemaphoreType.DMA,
        pltpu.SemaphoreType.DMA,
        pltpu.SemaphoreType.DMA,
    ]
    halves = [
        pl.kernel(
            _make_sc_gather(h),
            out_type=[
                jax.ShapeDtypeStruct((BH, D), jnp.float32),
                jax.ShapeDtypeStruct((BH, D), jnp.float32),
            ],
            mesh=mesh,
            scratch_types=scratch,
        )(node3d, attr3d, struct_table, attr_lut)
        for h in range(H)
    ]

    BLK = 2048
    nblk = BH // BLK
    semb0, acont0 = halves[0]
    out = pl.pallas_call(
        _fuse_body,
        grid=(nblk,),
        in_specs=[
            pl.BlockSpec((BLK, D), lambda i: (i, 0)),
            pl.BlockSpec((2 * D, D), lambda i: (0, 0)),
            pl.BlockSpec((BLK, D), lambda i: (i, 0)),
        ],
        out_specs=pl.BlockSpec((BLK, D), lambda i: (i, 0)),
        out_shape=jax.ShapeDtypeStruct((B, D), jnp.float32),
    )(semb0, fusion_w, acont0)
    for h in range(1, H):
        semb_h, acont_h = halves[h]
        out = pl.pallas_call(
            _fuse_body_aliased,
            grid=(nblk,),
            in_specs=[
                pl.BlockSpec(memory_space=pltpu.MemorySpace.HBM),
                pl.BlockSpec((BLK, D), lambda i: (i, 0)),
                pl.BlockSpec((2 * D, D), lambda i: (0, 0)),
                pl.BlockSpec((BLK, D), lambda i: (i, 0)),
            ],
            out_specs=pl.BlockSpec((BLK, D), lambda i, h=h: (i + h * nblk, 0)),
            out_shape=jax.ShapeDtypeStruct((B, D), jnp.float32),
            input_output_aliases={0: 0},
        )(out, semb_h, fusion_w, acont_h)
    return out
